# baseline (speedup 1.0000x reference)
import functools

import jax
import jax.numpy as jnp
from jax import lax
from jax.experimental import pallas as pl
from jax.experimental.pallas import tpu as pltpu
from jax.experimental.pallas import tpu_sc as plsc

_EMBED = 64
_BATCH = 16384
_NC = 2
_NS = 16
_NW = _NC * _NS
_BPW = _BATCH // _NW
_CH = 128
_NCH = _BPW // _CH


@functools.partial(
    pl.kernel,
    out_type=(
        jax.ShapeDtypeStruct((_BATCH, _EMBED), jnp.float32),
        jax.ShapeDtypeStruct((_BATCH, _EMBED), jnp.float32),
        jax.ShapeDtypeStruct((_BATCH, _EMBED), jnp.float32),
    ),
    mesh=plsc.VectorSubcoreMesh(core_axis_name="c", subcore_axis_name="s"),
    compiler_params=pltpu.CompilerParams(use_tc_tiling_on_sc=False),
    scratch_types=[
        pltpu.VMEM((_NCH, _CH), jnp.int32),
        pltpu.VMEM((_NCH, _CH), jnp.int32),
        pltpu.VMEM((_NCH, _CH), jnp.int32),
        pltpu.VMEM((_BPW, _EMBED), jnp.float32),
        pltpu.VMEM((_BPW, _EMBED), jnp.float32),
        pltpu.VMEM((_BPW, _EMBED), jnp.float32),
        pltpu.SemaphoreType.DMA,
        pltpu.SemaphoreType.DMA,
        pltpu.SemaphoreType.DMA,
        pltpu.SemaphoreType.DMA,
    ],
)
def _hetero_embed(nids_u, nids_i, nids_c, w_u, w_i, w_c,
                  out_u, out_i, out_c,
                  idx_u, idx_i, idx_c, rows_u, rows_i, rows_c,
                  sem_u, sem_i, sem_c, sem_out):
    wid = lax.axis_index("s") * _NC + lax.axis_index("c")
    base = wid * _BPW

    for j in range(_NCH):
        pltpu.sync_copy(nids_u.at[pl.ds(base + j * _CH, _CH)], idx_u.at[j])
        pltpu.sync_copy(nids_i.at[pl.ds(base + j * _CH, _CH)], idx_i.at[j])
        pltpu.sync_copy(nids_c.at[pl.ds(base + j * _CH, _CH)], idx_c.at[j])

    waits = []
    for j in range(_NCH):
        waits.append(pltpu.async_copy(
            w_u.at[idx_u.at[j]], rows_u.at[pl.ds(j * _CH, _CH)], sem_u))
        waits.append(pltpu.async_copy(
            w_i.at[idx_i.at[j]], rows_i.at[pl.ds(j * _CH, _CH)], sem_i))
        waits.append(pltpu.async_copy(
            w_c.at[idx_c.at[j]], rows_c.at[pl.ds(j * _CH, _CH)], sem_c))

    out_waits = []
    for k, (rows, out) in enumerate(((rows_u, out_u), (rows_i, out_i),
                                     (rows_c, out_c))):
        for j in range(_NCH):
            waits[j * 3 + k].wait()
        out_waits.append(pltpu.async_copy(
            rows, out.at[pl.ds(base, _BPW)], sem_out))
    for w in out_waits:
        w.wait()


def kernel(nids_user, nids_item, nids_cat, W_user, W_item, W_cat):
    return _hetero_embed(nids_user.astype(jnp.int32),
                         nids_item.astype(jnp.int32),
                         nids_cat.astype(jnp.int32),
                         W_user, W_item, W_cat)
